# Initial kernel scaffold; baseline (speedup 1.0000x reference)
import jax
import jax.numpy as jnp
from jax.experimental import pallas as pl


def kernel(token_inputs, w_gate, expert_capacity):
    raise NotImplementedError("write your pallas kernel here")


# speedup vs baseline: 1.7911x; 1.7911x over previous
import functools

import jax
import jax.numpy as jnp
from jax import lax
from jax.experimental import pallas as pl
from jax.experimental.pallas import tpu as pltpu
from jax.experimental.pallas import tpu_sc as plsc

DIM = 2048
NUM_EXPERTS = 64
BN = 512


def _sum64(x):
    p = x[:, 0:8]
    for k in range(1, 8):
        p = p + x[:, 8 * k:8 * k + 8]
    p = p[:, :4] + p[:, 4:]
    p = p[:, :2] + p[:, 2:]
    p = p[:, :1] + p[:, 1:]
    return p


def _tc_body(x_ref, w_ref, idx1_ref, idx2_ref, idxf1_ref, idxf2_ref,
             w1_ref, w2_ref, z_ref, aux_ref,
             zacc, auxacc, cntacc, psacc):
    b = pl.program_id(0)
    j = pl.program_id(1)
    nb = pl.num_programs(0)
    nj = pl.num_programs(1)
    E = NUM_EXPERTS

    x = x_ref[0]
    logits = jnp.dot(x, w_ref[...], preferred_element_type=jnp.float32)

    m = jnp.max(logits, axis=-1, keepdims=True)
    ex = jnp.exp(logits - m)
    s = _sum64(ex)
    lse = m + jnp.log(s)
    p1 = ex / s
    m2 = jnp.max(p1, axis=-1, keepdims=True)
    ex2 = jnp.exp(p1 - m2)
    probs = ex2 / _sum64(ex2)

    col = lax.broadcasted_iota(jnp.int32, probs.shape, 1)
    mx1 = jnp.max(probs, axis=-1, keepdims=True)
    idx1 = jnp.min(jnp.where(probs == mx1, col, E), axis=-1)
    masked = jnp.where(col == idx1[:, None], -1.0, probs)
    mx2 = jnp.max(masked, axis=-1, keepdims=True)
    idx2 = jnp.min(jnp.where(masked == mx2, col, E), axis=-1)

    idx1_ref[0, 0, :] = idx1
    idx2_ref[0, 0, :] = idx2
    idxf1_ref[0, 0, :] = idx1.astype(jnp.float32)
    idxf2_ref[0, 0, :] = idx2.astype(jnp.float32)
    w1_ref[0, 0, :] = mx1[:, 0]
    w2_ref[0, 0, :] = mx2[:, 0]

    z_blk = jnp.sum(lse[:, 0] ** 2)
    cnt_blk = jnp.sum(jnp.where(col == idx1[:, None], 1.0, 0.0)
                      + jnp.where(col == idx2[:, None], 1.0, 0.0), axis=0)
    ps_blk = jnp.sum(probs, axis=0)

    first = jnp.logical_and(b == 0, j == 0)

    @pl.when(first)
    def _():
        zacc[0, 0] = z_blk

    @pl.when(jnp.logical_not(first))
    def _():
        zacc[0, 0] += z_blk

    @pl.when(j == 0)
    def _():
        cntacc[0, :] = cnt_blk
        psacc[0, :] = ps_blk

    @pl.when(j > 0)
    def _():
        cntacc[0, :] += cnt_blk
        psacc[0, :] += ps_blk

    @pl.when(j == nj - 1)
    def _():
        aux_b = jnp.sum(cntacc[0, :] * psacc[0, :])

        @pl.when(b == 0)
        def _():
            auxacc[0, 0] = aux_b

        @pl.when(b > 0)
        def _():
            auxacc[0, 0] += aux_b

    @pl.when(jnp.logical_and(b == nb - 1, j == nj - 1))
    def _():
        n_tok = nb * nj * BN
        n_per_b = nj * BN
        z_ref[0, 0] = zacc[0, 0] / n_tok
        aux_ref[0, 0] = auxacc[0, 0] * E / (nb * n_per_b * n_per_b)


def _tc_router(tokens, w_gate):
    B, N, D = tokens.shape
    E = w_gate.shape[1]
    nj = N // BN
    nblk = B * nj
    row3 = lambda dt: jax.ShapeDtypeStruct((nblk, 1, BN), dt)
    rowspec = pl.BlockSpec((1, 1, BN), lambda b, j: (b * nj + j, 0, 0))
    sspec = pl.BlockSpec((1, 1), lambda b, j: (0, 0),
                         memory_space=pltpu.SMEM)
    return pl.pallas_call(
        _tc_body,
        grid=(B, nj),
        in_specs=[
            pl.BlockSpec((1, BN, D), lambda b, j: (b, j, 0)),
            pl.BlockSpec((D, E), lambda b, j: (0, 0)),
        ],
        out_specs=[rowspec, rowspec, rowspec, rowspec, rowspec, rowspec,
                   sspec, sspec],
        out_shape=[row3(jnp.int32), row3(jnp.int32), row3(jnp.float32),
                   row3(jnp.float32), row3(jnp.float32), row3(jnp.float32),
                   jax.ShapeDtypeStruct((1, 1), jnp.float32),
                   jax.ShapeDtypeStruct((1, 1), jnp.float32)],
        scratch_shapes=[
            pltpu.SMEM((1, 1), jnp.float32),
            pltpu.SMEM((1, 1), jnp.float32),
            pltpu.VMEM((1, E), jnp.float32),
            pltpu.VMEM((1, E), jnp.float32),
        ],
        compiler_params=pltpu.CompilerParams(
            dimension_semantics=("arbitrary", "arbitrary")),
    )(tokens, w_gate)


def _sc_priority(idx1, idx2, w1, w2, cap_vec):
    B, N = idx1.shape
    E = NUM_EXPERTS
    mesh = plsc.VectorSubcoreMesh(core_axis_name="c", subcore_axis_name="s")
    out = jax.ShapeDtypeStruct((B, N), jnp.float32)

    @functools.partial(
        pl.kernel,
        out_type=[out, out, out, out],
        mesh=mesh,
        scratch_types=[
            pltpu.VMEM((N,), jnp.int32), pltpu.VMEM((N,), jnp.int32),
            pltpu.VMEM((N,), jnp.float32), pltpu.VMEM((N,), jnp.float32),
            pltpu.VMEM((16,), jnp.int32), pltpu.VMEM((E,), jnp.int32),
            pltpu.VMEM((N,), jnp.float32), pltpu.VMEM((N,), jnp.float32),
            pltpu.VMEM((N,), jnp.float32), pltpu.VMEM((N,), jnp.float32),
        ],
        compiler_params=pltpu.CompilerParams(needs_layout_passes=False),
    )
    def sck(idx1_h, idx2_h, w1_h, w2_h, cap_h,
            pri1_h, pri2_h, c1_h, c2_h,
            idx1_v, idx2_v, w1_v, w2_v, cap_v, cnt_v,
            p1_v, p2_v, c1_v, c2_v):
        wid = lax.axis_index("s") * 2 + lax.axis_index("c")

        @pl.when(wid < B)
        def _():
            b = wid
            pltpu.sync_copy(idx1_h.at[b], idx1_v)
            pltpu.sync_copy(idx2_h.at[b], idx2_v)
            pltpu.sync_copy(w1_h.at[b], w1_v)
            pltpu.sync_copy(w2_h.at[b], w2_v)
            pltpu.sync_copy(cap_h, cap_v)
            capv = cap_v[...]
            zero16 = jnp.zeros((16,), jnp.int32)
            for i in range(E // 16):
                cnt_v[pl.ds(i * 16, 16)] = zero16
            rconst, _ = plsc.scan_count(zero16)
            base = rconst - lax.iota(jnp.int32, 16)

            def step(i, idx_v, w_v, pri_v, comb_v):
                ii = idx_v[pl.ds(i * 16, 16)]
                g = plsc.load_gather(cnt_v, [ii])
                r, lastm = plsc.scan_count(ii)
                pri = g + r - base
                plsc.store_scatter(cnt_v, [ii], pri + 1, mask=lastm)
                pri_v[pl.ds(i * 16, 16)] = pri.astype(jnp.float32)
                wl = w_v[pl.ds(i * 16, 16)]
                comb_v[pl.ds(i * 16, 16)] = jnp.where(pri < capv, wl, 0.0)

            def body1(i, c):
                step(i, idx1_v, w1_v, p1_v, c1_v)
                return c

            def body2(i, c):
                step(i, idx2_v, w2_v, p2_v, c2_v)
                return c

            lax.fori_loop(0, N // 16, body1, 0)
            lax.fori_loop(0, N // 16, body2, 0)
            pltpu.sync_copy(p1_v, pri1_h.at[b])
            pltpu.sync_copy(p2_v, pri2_h.at[b])
            pltpu.sync_copy(c1_v, c1_h.at[b])
            pltpu.sync_copy(c2_v, c2_h.at[b])

    return sck(idx1, idx2, w1, w2, cap_vec)


def kernel(token_inputs, w_gate, expert_capacity):
    B, N, D = token_inputs.shape
    (idx1_3, idx2_3, idxf1_3, idxf2_3, w1_3, w2_3,
     z11, aux11) = _tc_router(token_inputs, w_gate)
    rs = lambda a: a.reshape(B, N)
    cap_vec = jnp.full((16,), expert_capacity, jnp.int32)
    pri1, pri2, c1, c2 = _sc_priority(rs(idx1_3), rs(idx2_3),
                                      rs(w1_3), rs(w2_3), cap_vec)
    dispatch = jnp.stack([rs(idxf1_3), rs(idxf2_3), pri1, pri2], axis=-1)
    combine = jnp.stack([c1, c2], axis=-1)
    return dispatch, combine, aux11[0, 0], z11[0, 0]


# speedup vs baseline: 3.4501x; 1.9262x over previous
import functools

import jax
import jax.numpy as jnp
from jax import lax
from jax.experimental import pallas as pl
from jax.experimental.pallas import tpu as pltpu
from jax.experimental.pallas import tpu_sc as plsc

DIM = 2048
NUM_EXPERTS = 64
BN = 512
LANES = 128


def _sum64(x):
    p = x[0:8]
    for k in range(1, 8):
        p = p + x[8 * k:8 * k + 8]
    p = p[0:4] + p[4:8]
    p = p[0:2] + p[2:4]
    p = p[0:1] + p[1:2]
    return p


def _tc_body(x_ref, wt_ref, idx1_ref, idx2_ref, idxf1_ref, idxf2_ref,
             w1_ref, w2_ref, z_ref, aux_ref,
             zacc, auxacc, zrow, cntacc, psacc):
    b = pl.program_id(0)
    j = pl.program_id(1)
    nb = pl.num_programs(0)
    nj = pl.num_programs(1)
    E = NUM_EXPERTS

    x = x_ref[0]
    logits = lax.dot_general(wt_ref[...], x, (((1,), (1,)), ((), ())),
                             preferred_element_type=jnp.float32)

    m = jnp.max(logits, axis=0, keepdims=True)
    ex = jnp.exp(logits - m)
    s = _sum64(ex)
    lse = m + jnp.log(s)
    p1 = ex / s
    m2 = 1.0 / s
    ex2 = jnp.exp(p1 - m2)
    s2 = _sum64(ex2)
    probs = ex2 / s2
    mx1 = 1.0 / s2

    row = lax.broadcasted_iota(jnp.int32, probs.shape, 0)
    idx1 = jnp.min(jnp.where(probs == mx1, row, E), axis=0)
    masked = jnp.where(row == idx1[None, :], -1.0, probs)
    mx2 = jnp.max(masked, axis=0, keepdims=True)
    idx2 = jnp.min(jnp.where(masked == mx2, row, E), axis=0)

    idx1_ref[0, 0, :] = idx1
    idx2_ref[0, 0, :] = idx2
    idxf1_ref[0, 0, :] = idx1.astype(jnp.float32)
    idxf2_ref[0, 0, :] = idx2.astype(jnp.float32)
    w1_ref[0, 0, :] = mx1[0]
    w2_ref[0, 0, :] = mx2[0]

    first = jnp.logical_and(b == 0, j == 0)

    @pl.when(first)
    def _():
        zrow[...] = jnp.zeros_like(zrow)

    zrow[...] += lse * lse

    onehot = (jnp.where(row == idx1[None, :], 1.0, 0.0)
              + jnp.where(row == idx2[None, :], 1.0, 0.0))
    cnt_c = onehot[:, 0:LANES]
    ps_c = probs[:, 0:LANES]
    for c in range(1, BN // LANES):
        cnt_c = cnt_c + onehot[:, c * LANES:(c + 1) * LANES]
        ps_c = ps_c + probs[:, c * LANES:(c + 1) * LANES]

    @pl.when(j == 0)
    def _():
        cntacc[...] = cnt_c
        psacc[...] = ps_c

    @pl.when(j > 0)
    def _():
        cntacc[...] += cnt_c
        psacc[...] += ps_c

    @pl.when(j == nj - 1)
    def _():
        cnt_tot = jnp.sum(cntacc[...], axis=1)
        ps_tot = jnp.sum(psacc[...], axis=1)
        aux_b = jnp.sum(cnt_tot * ps_tot)

        @pl.when(b == 0)
        def _():
            auxacc[0, 0] = aux_b

        @pl.when(b > 0)
        def _():
            auxacc[0, 0] += aux_b

    @pl.when(jnp.logical_and(b == nb - 1, j == nj - 1))
    def _():
        n_tok = nb * nj * BN
        n_per_b = nj * BN
        z_ref[0, 0] = jnp.sum(zrow[...]) / n_tok
        aux_ref[0, 0] = auxacc[0, 0] * E / (nb * n_per_b * n_per_b)


def _tc_router(tokens, w_gate_t):
    B, N, D = tokens.shape
    E = w_gate_t.shape[0]
    nj = N // BN
    nblk = B * nj
    row3 = lambda dt: jax.ShapeDtypeStruct((nblk, 1, BN), dt)
    rowspec = pl.BlockSpec((1, 1, BN), lambda b, j: (b * nj + j, 0, 0))
    sspec = pl.BlockSpec((1, 1), lambda b, j: (0, 0),
                         memory_space=pltpu.SMEM)
    return pl.pallas_call(
        _tc_body,
        grid=(B, nj),
        in_specs=[
            pl.BlockSpec((1, BN, D), lambda b, j: (b, j, 0)),
            pl.BlockSpec((E, D), lambda b, j: (0, 0)),
        ],
        out_specs=[rowspec, rowspec, rowspec, rowspec, rowspec, rowspec,
                   sspec, sspec],
        out_shape=[row3(jnp.int32), row3(jnp.int32), row3(jnp.float32),
                   row3(jnp.float32), row3(jnp.float32), row3(jnp.float32),
                   jax.ShapeDtypeStruct((1, 1), jnp.float32),
                   jax.ShapeDtypeStruct((1, 1), jnp.float32)],
        scratch_shapes=[
            pltpu.SMEM((1, 1), jnp.float32),
            pltpu.SMEM((1, 1), jnp.float32),
            pltpu.VMEM((1, BN), jnp.float32),
            pltpu.VMEM((NUM_EXPERTS, LANES), jnp.float32),
            pltpu.VMEM((NUM_EXPERTS, LANES), jnp.float32),
        ],
        compiler_params=pltpu.CompilerParams(
            dimension_semantics=("arbitrary", "arbitrary")),
    )(tokens, w_gate_t)


def _sc_priority(idx1, idx2, w1, w2, cap_vec):
    B, N = idx1.shape
    E = NUM_EXPERTS
    mesh = plsc.VectorSubcoreMesh(core_axis_name="c", subcore_axis_name="s")
    out = jax.ShapeDtypeStruct((B, N), jnp.float32)

    @functools.partial(
        pl.kernel,
        out_type=[out, out, out, out],
        mesh=mesh,
        scratch_types=[
            pltpu.VMEM((N,), jnp.int32), pltpu.VMEM((N,), jnp.int32),
            pltpu.VMEM((N,), jnp.float32), pltpu.VMEM((N,), jnp.float32),
            pltpu.VMEM((16,), jnp.int32), pltpu.VMEM((E,), jnp.int32),
            pltpu.VMEM((N,), jnp.float32), pltpu.VMEM((N,), jnp.float32),
            pltpu.VMEM((N,), jnp.float32), pltpu.VMEM((N,), jnp.float32),
        ],
        compiler_params=pltpu.CompilerParams(needs_layout_passes=False),
    )
    def sck(idx1_h, idx2_h, w1_h, w2_h, cap_h,
            pri1_h, pri2_h, c1_h, c2_h,
            idx1_v, idx2_v, w1_v, w2_v, cap_v, cnt_v,
            p1_v, p2_v, c1_v, c2_v):
        wid = lax.axis_index("s") * 2 + lax.axis_index("c")

        @pl.when(wid < B)
        def _():
            b = wid
            pltpu.sync_copy(idx1_h.at[b], idx1_v)
            pltpu.sync_copy(idx2_h.at[b], idx2_v)
            pltpu.sync_copy(w1_h.at[b], w1_v)
            pltpu.sync_copy(w2_h.at[b], w2_v)
            pltpu.sync_copy(cap_h, cap_v)
            capv = cap_v[...]
            zero16 = jnp.zeros((16,), jnp.int32)
            for i in range(E // 16):
                cnt_v[pl.ds(i * 16, 16)] = zero16
            rconst, _ = plsc.scan_count(zero16)
            base = rconst - lax.iota(jnp.int32, 16)

            def step(i, idx_v, w_v, pri_v, comb_v):
                ii = idx_v[pl.ds(i * 16, 16)]
                g = plsc.load_gather(cnt_v, [ii])
                r, lastm = plsc.scan_count(ii)
                pri = g + r - base
                plsc.store_scatter(cnt_v, [ii], pri + 1, mask=lastm)
                pri_v[pl.ds(i * 16, 16)] = pri.astype(jnp.float32)
                wl = w_v[pl.ds(i * 16, 16)]
                comb_v[pl.ds(i * 16, 16)] = jnp.where(pri < capv, wl, 0.0)

            def body1(i, c):
                step(i, idx1_v, w1_v, p1_v, c1_v)
                return c

            def body2(i, c):
                step(i, idx2_v, w2_v, p2_v, c2_v)
                return c

            lax.fori_loop(0, N // 16, body1, 0)
            lax.fori_loop(0, N // 16, body2, 0)
            pltpu.sync_copy(p1_v, pri1_h.at[b])
            pltpu.sync_copy(p2_v, pri2_h.at[b])
            pltpu.sync_copy(c1_v, c1_h.at[b])
            pltpu.sync_copy(c2_v, c2_h.at[b])

    return sck(idx1, idx2, w1, w2, cap_vec)


def kernel(token_inputs, w_gate, expert_capacity):
    B, N, D = token_inputs.shape
    (idx1_3, idx2_3, idxf1_3, idxf2_3, w1_3, w2_3,
     z11, aux11) = _tc_router(token_inputs, w_gate.T)
    rs = lambda a: a.reshape(B, N)
    cap_vec = jnp.full((16,), expert_capacity, jnp.int32)
    pri1, pri2, c1, c2 = _sc_priority(rs(idx1_3), rs(idx2_3),
                                      rs(w1_3), rs(w2_3), cap_vec)
    dispatch = jnp.stack([rs(idxf1_3), rs(idxf2_3), pri1, pri2], axis=-1)
    combine = jnp.stack([c1, c2], axis=-1)
    return dispatch, combine, aux11[0, 0], z11[0, 0]


# speedup vs baseline: 3.7162x; 1.0771x over previous
import functools

import jax
import jax.numpy as jnp
from jax import lax
from jax.experimental import pallas as pl
from jax.experimental.pallas import tpu as pltpu
from jax.experimental.pallas import tpu_sc as plsc

DIM = 2048
NUM_EXPERTS = 64
BN = 1024
LANES = 128


def _sum64(x):
    p = x[0:8]
    for k in range(1, 8):
        p = p + x[8 * k:8 * k + 8]
    p = p[0:4] + p[4:8]
    p = p[0:2] + p[2:4]
    p = p[0:1] + p[1:2]
    return p


def _tc_body(x_ref, wt_ref, idx1_ref, idx2_ref, idxf1_ref, idxf2_ref,
             w1_ref, w2_ref, z_ref, aux_ref,
             zacc, auxacc, zrow, cntacc, psacc):
    b = pl.program_id(0)
    j = pl.program_id(1)
    nb = pl.num_programs(0)
    nj = pl.num_programs(1)
    E = NUM_EXPERTS

    x = x_ref[0]
    logits = lax.dot_general(wt_ref[...], x, (((1,), (1,)), ((), ())),
                             preferred_element_type=jnp.float32)

    m = jnp.max(logits, axis=0, keepdims=True)
    ex = jnp.exp(logits - m)
    s = _sum64(ex)
    lse = m + jnp.log(s)
    p1 = ex / s
    m2 = 1.0 / s
    ex2 = jnp.exp(p1 - m2)
    s2 = _sum64(ex2)
    probs = ex2 / s2
    mx1 = 1.0 / s2

    row = lax.broadcasted_iota(jnp.int32, probs.shape, 0)
    idx1 = jnp.min(jnp.where(probs == mx1, row, E), axis=0)
    masked = jnp.where(row == idx1[None, :], -1.0, probs)
    mx2 = jnp.max(masked, axis=0, keepdims=True)
    idx2 = jnp.min(jnp.where(masked == mx2, row, E), axis=0)

    idx1_ref[0, 0, :] = idx1
    idx2_ref[0, 0, :] = idx2
    idxf1_ref[0, 0, :] = idx1.astype(jnp.float32)
    idxf2_ref[0, 0, :] = idx2.astype(jnp.float32)
    w1_ref[0, 0, :] = mx1[0]
    w2_ref[0, 0, :] = mx2[0]

    first = jnp.logical_and(b == 0, j == 0)

    @pl.when(first)
    def _():
        zrow[...] = jnp.zeros_like(zrow)

    zrow[...] += lse * lse

    onehot = (jnp.where(row == idx1[None, :], 1.0, 0.0)
              + jnp.where(row == idx2[None, :], 1.0, 0.0))
    cnt_c = onehot[:, 0:LANES]
    ps_c = probs[:, 0:LANES]
    for c in range(1, BN // LANES):
        cnt_c = cnt_c + onehot[:, c * LANES:(c + 1) * LANES]
        ps_c = ps_c + probs[:, c * LANES:(c + 1) * LANES]

    @pl.when(j == 0)
    def _():
        cntacc[...] = cnt_c
        psacc[...] = ps_c

    @pl.when(j > 0)
    def _():
        cntacc[...] += cnt_c
        psacc[...] += ps_c

    @pl.when(j == nj - 1)
    def _():
        cnt_tot = jnp.sum(cntacc[...], axis=1)
        ps_tot = jnp.sum(psacc[...], axis=1)
        aux_b = jnp.sum(cnt_tot * ps_tot)

        @pl.when(b == 0)
        def _():
            auxacc[0, 0] = aux_b

        @pl.when(b > 0)
        def _():
            auxacc[0, 0] += aux_b

    @pl.when(jnp.logical_and(b == nb - 1, j == nj - 1))
    def _():
        n_tok = nb * nj * BN
        n_per_b = nj * BN
        z_ref[0, 0] = jnp.sum(zrow[...]) / n_tok
        aux_ref[0, 0] = auxacc[0, 0] * E / (nb * n_per_b * n_per_b)


def _tc_router(tokens, w_gate_t):
    B, N, D = tokens.shape
    E = w_gate_t.shape[0]
    nj = N // BN
    nblk = B * nj
    row3 = lambda dt: jax.ShapeDtypeStruct((nblk, 1, BN), dt)
    rowspec = pl.BlockSpec((1, 1, BN), lambda b, j: (b * nj + j, 0, 0))
    sspec = pl.BlockSpec((1, 1), lambda b, j: (0, 0),
                         memory_space=pltpu.SMEM)
    return pl.pallas_call(
        _tc_body,
        grid=(B, nj),
        in_specs=[
            pl.BlockSpec((1, BN, D), lambda b, j: (b, j, 0)),
            pl.BlockSpec((E, D), lambda b, j: (0, 0)),
        ],
        out_specs=[rowspec, rowspec, rowspec, rowspec, rowspec, rowspec,
                   sspec, sspec],
        out_shape=[row3(jnp.int32), row3(jnp.int32), row3(jnp.float32),
                   row3(jnp.float32), row3(jnp.float32), row3(jnp.float32),
                   jax.ShapeDtypeStruct((1, 1), jnp.float32),
                   jax.ShapeDtypeStruct((1, 1), jnp.float32)],
        scratch_shapes=[
            pltpu.SMEM((1, 1), jnp.float32),
            pltpu.SMEM((1, 1), jnp.float32),
            pltpu.VMEM((1, BN), jnp.float32),
            pltpu.VMEM((NUM_EXPERTS, LANES), jnp.float32),
            pltpu.VMEM((NUM_EXPERTS, LANES), jnp.float32),
        ],
        compiler_params=pltpu.CompilerParams(
            dimension_semantics=("arbitrary", "arbitrary")),
    )(tokens, w_gate_t)


def _sc_priority(idx1, idx2, w1, w2, cap_vec):
    B, N = idx1.shape
    E = NUM_EXPERTS
    mesh = plsc.VectorSubcoreMesh(core_axis_name="c", subcore_axis_name="s")
    out = jax.ShapeDtypeStruct((B, N), jnp.float32)

    @functools.partial(
        pl.kernel,
        out_type=[out, out, out, out],
        mesh=mesh,
        scratch_types=[
            pltpu.VMEM((N,), jnp.int32), pltpu.VMEM((N,), jnp.int32),
            pltpu.VMEM((N,), jnp.float32), pltpu.VMEM((N,), jnp.float32),
            pltpu.VMEM((16,), jnp.int32), pltpu.VMEM((E,), jnp.int32),
            pltpu.VMEM((N,), jnp.float32), pltpu.VMEM((N,), jnp.float32),
            pltpu.VMEM((N,), jnp.float32), pltpu.VMEM((N,), jnp.float32),
        ],
        compiler_params=pltpu.CompilerParams(needs_layout_passes=False),
    )
    def sck(idx1_h, idx2_h, w1_h, w2_h, cap_h,
            pri1_h, pri2_h, c1_h, c2_h,
            idx1_v, idx2_v, w1_v, w2_v, cap_v, cnt_v,
            p1_v, p2_v, c1_v, c2_v):
        wid = lax.axis_index("s") * 2 + lax.axis_index("c")

        @pl.when(wid < B)
        def _():
            b = wid
            pltpu.sync_copy(idx1_h.at[b], idx1_v)
            pltpu.sync_copy(idx2_h.at[b], idx2_v)
            pltpu.sync_copy(w1_h.at[b], w1_v)
            pltpu.sync_copy(w2_h.at[b], w2_v)
            pltpu.sync_copy(cap_h, cap_v)
            capv = cap_v[...]
            zero16 = jnp.zeros((16,), jnp.int32)
            for i in range(E // 16):
                cnt_v[pl.ds(i * 16, 16)] = zero16
            rconst, _ = plsc.scan_count(zero16)
            base = rconst - lax.iota(jnp.int32, 16)

            def step(i, idx_v, w_v, pri_v, comb_v):
                ii = idx_v[pl.ds(i * 16, 16)]
                g = plsc.load_gather(cnt_v, [ii])
                r, lastm = plsc.scan_count(ii)
                pri = g + r - base
                plsc.store_scatter(cnt_v, [ii], pri + 1, mask=lastm)
                pri_v[pl.ds(i * 16, 16)] = pri.astype(jnp.float32)
                wl = w_v[pl.ds(i * 16, 16)]
                comb_v[pl.ds(i * 16, 16)] = jnp.where(pri < capv, wl, 0.0)

            def body1(i, c):
                step(i, idx1_v, w1_v, p1_v, c1_v)
                return c

            def body2(i, c):
                step(i, idx2_v, w2_v, p2_v, c2_v)
                return c

            lax.fori_loop(0, N // 16, body1, 0)
            lax.fori_loop(0, N // 16, body2, 0)
            pltpu.sync_copy(p1_v, pri1_h.at[b])
            pltpu.sync_copy(p2_v, pri2_h.at[b])
            pltpu.sync_copy(c1_v, c1_h.at[b])
            pltpu.sync_copy(c2_v, c2_h.at[b])

    return sck(idx1, idx2, w1, w2, cap_vec)


def kernel(token_inputs, w_gate, expert_capacity):
    B, N, D = token_inputs.shape
    (idx1_3, idx2_3, idxf1_3, idxf2_3, w1_3, w2_3,
     z11, aux11) = _tc_router(token_inputs, w_gate.T)
    rs = lambda a: a.reshape(B, N)
    cap_vec = jnp.full((16,), expert_capacity, jnp.int32)
    pri1, pri2, c1, c2 = _sc_priority(rs(idx1_3), rs(idx2_3),
                                      rs(w1_3), rs(w2_3), cap_vec)
    dispatch = jnp.stack([rs(idxf1_3), rs(idxf2_3), pri1, pri2], axis=-1)
    combine = jnp.stack([c1, c2], axis=-1)
    return dispatch, combine, aux11[0, 0], z11[0, 0]


# speedup vs baseline: 3.8867x; 1.0459x over previous
import functools

import jax
import jax.numpy as jnp
from jax import lax
from jax.experimental import pallas as pl
from jax.experimental.pallas import tpu as pltpu
from jax.experimental.pallas import tpu_sc as plsc

DIM = 2048
NUM_EXPERTS = 64
BN = 2048
LANES = 128


def _sum64(x):
    p = x[0:8]
    for k in range(1, 8):
        p = p + x[8 * k:8 * k + 8]
    p = p[0:4] + p[4:8]
    p = p[0:2] + p[2:4]
    p = p[0:1] + p[1:2]
    return p


def _tc_body(x_ref, wt_ref, idx1_ref, idx2_ref, idxf1_ref, idxf2_ref,
             w1_ref, w2_ref, z_ref, aux_ref,
             zacc, auxacc, zrow, cntacc, psacc):
    b = pl.program_id(0)
    j = pl.program_id(1)
    nb = pl.num_programs(0)
    nj = pl.num_programs(1)
    E = NUM_EXPERTS

    x = x_ref[0]
    logits = lax.dot_general(wt_ref[...], x, (((1,), (1,)), ((), ())),
                             preferred_element_type=jnp.float32)

    m = jnp.max(logits, axis=0, keepdims=True)
    ex = jnp.exp(logits - m)
    s = _sum64(ex)
    lse = m + jnp.log(s)
    p1 = ex / s
    m2 = 1.0 / s
    ex2 = jnp.exp(p1 - m2)
    s2 = _sum64(ex2)
    probs = ex2 / s2
    mx1 = 1.0 / s2

    row = lax.broadcasted_iota(jnp.int32, probs.shape, 0)
    idx1 = jnp.min(jnp.where(probs == mx1, row, E), axis=0)
    masked = jnp.where(row == idx1[None, :], -1.0, probs)
    mx2 = jnp.max(masked, axis=0, keepdims=True)
    idx2 = jnp.min(jnp.where(masked == mx2, row, E), axis=0)

    idx1_ref[0, 0, :] = idx1
    idx2_ref[0, 0, :] = idx2
    idxf1_ref[0, 0, :] = idx1.astype(jnp.float32)
    idxf2_ref[0, 0, :] = idx2.astype(jnp.float32)
    w1_ref[0, 0, :] = mx1[0]
    w2_ref[0, 0, :] = mx2[0]

    first = jnp.logical_and(b == 0, j == 0)

    @pl.when(first)
    def _():
        zrow[...] = jnp.zeros_like(zrow)

    zrow[...] += lse * lse

    onehot = (jnp.where(row == idx1[None, :], 1.0, 0.0)
              + jnp.where(row == idx2[None, :], 1.0, 0.0))
    cnt_c = onehot[:, 0:LANES]
    ps_c = probs[:, 0:LANES]
    for c in range(1, BN // LANES):
        cnt_c = cnt_c + onehot[:, c * LANES:(c + 1) * LANES]
        ps_c = ps_c + probs[:, c * LANES:(c + 1) * LANES]

    @pl.when(j == 0)
    def _():
        cntacc[...] = cnt_c
        psacc[...] = ps_c

    @pl.when(j > 0)
    def _():
        cntacc[...] += cnt_c
        psacc[...] += ps_c

    @pl.when(j == nj - 1)
    def _():
        cnt_tot = jnp.sum(cntacc[...], axis=1)
        ps_tot = jnp.sum(psacc[...], axis=1)
        aux_b = jnp.sum(cnt_tot * ps_tot)

        @pl.when(b == 0)
        def _():
            auxacc[0, 0] = aux_b

        @pl.when(b > 0)
        def _():
            auxacc[0, 0] += aux_b

    @pl.when(jnp.logical_and(b == nb - 1, j == nj - 1))
    def _():
        n_tok = nb * nj * BN
        n_per_b = nj * BN
        z_ref[0, 0] = jnp.sum(zrow[...]) / n_tok
        aux_ref[0, 0] = auxacc[0, 0] * E / (nb * n_per_b * n_per_b)


def _tc_router(tokens, w_gate_t):
    B, N, D = tokens.shape
    E = w_gate_t.shape[0]
    nj = N // BN
    nblk = B * nj
    row3 = lambda dt: jax.ShapeDtypeStruct((nblk, 1, BN), dt)
    rowspec = pl.BlockSpec((1, 1, BN), lambda b, j: (b * nj + j, 0, 0))
    sspec = pl.BlockSpec((1, 1), lambda b, j: (0, 0),
                         memory_space=pltpu.SMEM)
    return pl.pallas_call(
        _tc_body,
        grid=(B, nj),
        in_specs=[
            pl.BlockSpec((1, BN, D), lambda b, j: (b, j, 0)),
            pl.BlockSpec((E, D), lambda b, j: (0, 0)),
        ],
        out_specs=[rowspec, rowspec, rowspec, rowspec, rowspec, rowspec,
                   sspec, sspec],
        out_shape=[row3(jnp.int32), row3(jnp.int32), row3(jnp.float32),
                   row3(jnp.float32), row3(jnp.float32), row3(jnp.float32),
                   jax.ShapeDtypeStruct((1, 1), jnp.float32),
                   jax.ShapeDtypeStruct((1, 1), jnp.float32)],
        scratch_shapes=[
            pltpu.SMEM((1, 1), jnp.float32),
            pltpu.SMEM((1, 1), jnp.float32),
            pltpu.VMEM((1, BN), jnp.float32),
            pltpu.VMEM((NUM_EXPERTS, LANES), jnp.float32),
            pltpu.VMEM((NUM_EXPERTS, LANES), jnp.float32),
        ],
        compiler_params=pltpu.CompilerParams(
            dimension_semantics=("arbitrary", "arbitrary")),
    )(tokens, w_gate_t)


def _sc_priority(idx1, idx2, w1, w2, cap_vec):
    B, N = idx1.shape
    E = NUM_EXPERTS
    mesh = plsc.VectorSubcoreMesh(core_axis_name="c", subcore_axis_name="s")
    out = jax.ShapeDtypeStruct((B, N), jnp.float32)

    @functools.partial(
        pl.kernel,
        out_type=[out, out, out, out],
        mesh=mesh,
        scratch_types=[
            pltpu.VMEM((N,), jnp.int32), pltpu.VMEM((N,), jnp.int32),
            pltpu.VMEM((N,), jnp.float32), pltpu.VMEM((N,), jnp.float32),
            pltpu.VMEM((16,), jnp.int32), pltpu.VMEM((E,), jnp.int32),
            pltpu.VMEM((N,), jnp.float32), pltpu.VMEM((N,), jnp.float32),
            pltpu.VMEM((N,), jnp.float32), pltpu.VMEM((N,), jnp.float32),
        ],
        compiler_params=pltpu.CompilerParams(needs_layout_passes=False),
    )
    def sck(idx1_h, idx2_h, w1_h, w2_h, cap_h,
            pri1_h, pri2_h, c1_h, c2_h,
            idx1_v, idx2_v, w1_v, w2_v, cap_v, cnt_v,
            p1_v, p2_v, c1_v, c2_v):
        wid = lax.axis_index("s") * 2 + lax.axis_index("c")

        @pl.when(wid < B)
        def _():
            b = wid
            pltpu.sync_copy(idx1_h.at[b], idx1_v)
            pltpu.sync_copy(idx2_h.at[b], idx2_v)
            pltpu.sync_copy(w1_h.at[b], w1_v)
            pltpu.sync_copy(w2_h.at[b], w2_v)
            pltpu.sync_copy(cap_h, cap_v)
            capv = cap_v[...]
            zero16 = jnp.zeros((16,), jnp.int32)
            for i in range(E // 16):
                cnt_v[pl.ds(i * 16, 16)] = zero16
            rconst, _ = plsc.scan_count(zero16)
            base = rconst - lax.iota(jnp.int32, 16)

            def step(i, idx_v, w_v, pri_v, comb_v):
                ii = idx_v[pl.ds(i * 16, 16)]
                g = plsc.load_gather(cnt_v, [ii])
                r, lastm = plsc.scan_count(ii)
                pri = g + r - base
                plsc.store_scatter(cnt_v, [ii], pri + 1, mask=lastm)
                pri_v[pl.ds(i * 16, 16)] = pri.astype(jnp.float32)
                wl = w_v[pl.ds(i * 16, 16)]
                comb_v[pl.ds(i * 16, 16)] = jnp.where(pri < capv, wl, 0.0)

            def body1(i, c):
                step(i, idx1_v, w1_v, p1_v, c1_v)
                return c

            def body2(i, c):
                step(i, idx2_v, w2_v, p2_v, c2_v)
                return c

            lax.fori_loop(0, N // 16, body1, 0)
            lax.fori_loop(0, N // 16, body2, 0)
            pltpu.sync_copy(p1_v, pri1_h.at[b])
            pltpu.sync_copy(p2_v, pri2_h.at[b])
            pltpu.sync_copy(c1_v, c1_h.at[b])
            pltpu.sync_copy(c2_v, c2_h.at[b])

    return sck(idx1, idx2, w1, w2, cap_vec)


def kernel(token_inputs, w_gate, expert_capacity):
    B, N, D = token_inputs.shape
    (idx1_3, idx2_3, idxf1_3, idxf2_3, w1_3, w2_3,
     z11, aux11) = _tc_router(token_inputs, w_gate.T)
    rs = lambda a: a.reshape(B, N)
    cap_vec = jnp.full((16,), expert_capacity, jnp.int32)
    pri1, pri2, c1, c2 = _sc_priority(rs(idx1_3), rs(idx2_3),
                                      rs(w1_3), rs(w2_3), cap_vec)
    dispatch = jnp.stack([rs(idxf1_3), rs(idxf2_3), pri1, pri2], axis=-1)
    combine = jnp.stack([c1, c2], axis=-1)
    return dispatch, combine, aux11[0, 0], z11[0, 0]
